# baseline (speedup 1.0000x reference)
import functools

import jax
import jax.numpy as jnp
from jax import lax
from jax.experimental import pallas as pl
from jax.experimental.pallas import tpu as pltpu
from jax.experimental.pallas import tpu_sc as plsc

D_MODEL = 64
SCALE = 8.0
NC, NS = 2, 16
NW = NC * NS
BB = 128
LANES = 16
TILE = 8 * BB


def kernel(x, table):
    B0, S = x.shape
    assert B0 == NW * BB and D_MODEL == table.shape[1]
    xt = x.T
    if xt.dtype != jnp.int32:
        xt = xt.astype(jnp.int32)

    mesh = plsc.VectorSubcoreMesh(core_axis_name="c", subcore_axis_name="s")

    @functools.partial(
        pl.kernel,
        mesh=mesh,
        out_type=jax.ShapeDtypeStruct((S, 8, NW, TILE), jnp.float32),
        scratch_types=[
            pltpu.VMEM((S, BB), jnp.int32),
            pltpu.VMEM((BB, D_MODEL), jnp.float32),
            pltpu.VMEM((BB, D_MODEL), jnp.float32),
            pltpu.VMEM((D_MODEL * BB,), jnp.float32),
            pltpu.VMEM((D_MODEL * BB,), jnp.float32),
            pltpu.SemaphoreType.DMA,
            pltpu.SemaphoreType.DMA,
            pltpu.SemaphoreType.DMA,
            pltpu.SemaphoreType.DMA,
        ],
        compiler_params=pltpu.CompilerParams(
            use_tc_tiling_on_sc=False, needs_layout_passes=False),
    )
    def emb(xt_hbm, table_hbm, out_hbm, idx_v, rows0, rows1, trans0, trans1,
            gsem0, gsem1, osem0, osem1):
        rows_bufs = (rows0, rows1)
        trans_bufs = (trans0, trans1)
        gsems = (gsem0, gsem1)
        osems = (osem0, osem1)
        wid = lax.axis_index("s") * NC + lax.axis_index("c")
        pltpu.sync_copy(xt_hbm.at[:, pl.ds(wid * BB, BB)], idx_v)
        for p in range(2):
            pltpu.async_copy(
                table_hbm.at[idx_v.at[p]], rows_bufs[p], gsems[p])

        iota = lax.iota(jnp.int32, LANES)
        chunk_base = [
            ((iota >> 3) + 2 * c) * TILE + (iota & 7) * BB
            for c in range(D_MODEL // LANES)
        ]

        def out_copies(s, p, wait):
            trans = trans_bufs[p]
            for td in range(8):
                cp = pltpu.make_async_copy(
                    trans.at[pl.ds(td * TILE, TILE)],
                    out_hbm.at[s, td, wid], osems[p])
                if wait:
                    cp.wait()
                else:
                    cp.start()

        def step(i, carry):
            for p in range(2):
                s = i * 2 + p
                rows = rows_bufs[p]
                trans = trans_bufs[p]
                pltpu.make_async_copy(
                    table_hbm.at[idx_v.at[s]], rows, gsems[p]).wait()

                @pl.when(s >= 2)
                def _(s=s, p=p):
                    out_copies(s - 2, p, wait=True)

                @plsc.parallel_loop(0, BB, unroll=4)
                def _(bi, rows=rows, trans=trans):
                    for c in range(D_MODEL // LANES):
                        vals = rows[bi, pl.ds(c * LANES, LANES)]
                        plsc.store_scatter(
                            trans, [chunk_base[c] + bi], vals * SCALE)

                out_copies(s, p, wait=False)

                @pl.when(s + 2 < S)
                def _(s=s, p=p, rows=rows):
                    pltpu.async_copy(
                        table_hbm.at[idx_v.at[s + 2]], rows, gsems[p])

            return carry

        lax.fori_loop(0, S // 2, step, 0)
        for p in range(2):
            out_copies(S - 2 + p, p, wait=True)

    out5 = emb(xt, table)
    out5 = out5.reshape(S, 8, NW, 8, BB)
    return out5.transpose(2, 4, 0, 1, 3).reshape(B0, S, D_MODEL)


# speedup vs baseline: 1.4748x; 1.4748x over previous
import functools

import jax
import jax.numpy as jnp
from jax import lax
from jax.experimental import pallas as pl
from jax.experimental.pallas import tpu as pltpu
from jax.experimental.pallas import tpu_sc as plsc

D_MODEL = 64
SCALE = 8.0
NC, NS = 2, 16
NW = NC * NS
BB = 128
LANES = 16
TILE = 8 * BB


def kernel(x, table):
    B0, S = x.shape
    V = table.shape[0]
    assert B0 == NW * BB and D_MODEL == table.shape[1]
    xt = x.T
    if xt.dtype != jnp.int32:
        xt = xt.astype(jnp.int32)
    tablep = jnp.pad(table, ((0, 0), (0, 2 * D_MODEL - table.shape[1])))

    mesh = plsc.VectorSubcoreMesh(core_axis_name="c", subcore_axis_name="s")

    @functools.partial(
        pl.kernel,
        mesh=mesh,
        out_type=jax.ShapeDtypeStruct((S, 8, NW, TILE), jnp.float32),
        scratch_types=[
            pltpu.VMEM((S, BB), jnp.int32),
            pltpu.VMEM((BB, 2 * D_MODEL), jnp.float32),
            pltpu.VMEM((BB, 2 * D_MODEL), jnp.float32),
            pltpu.VMEM((D_MODEL * BB,), jnp.float32),
            pltpu.VMEM((D_MODEL * BB,), jnp.float32),
            pltpu.SemaphoreType.DMA,
            pltpu.SemaphoreType.DMA,
            pltpu.SemaphoreType.DMA,
            pltpu.SemaphoreType.DMA,
        ],
        compiler_params=pltpu.CompilerParams(
            use_tc_tiling_on_sc=False, needs_layout_passes=False),
    )
    def emb(xt_hbm, tablep_hbm, out_hbm, idx_v, rows0, rows1, trans0, trans1,
            gsem0, gsem1, osem0, osem1):
        rows_bufs = (rows0, rows1)
        trans_bufs = (trans0, trans1)
        gsems = (gsem0, gsem1)
        osems = (osem0, osem1)
        wid = lax.axis_index("s") * NC + lax.axis_index("c")
        pltpu.sync_copy(xt_hbm.at[:, pl.ds(wid * BB, BB)], idx_v)

        iota = lax.iota(jnp.int32, LANES)
        dcol = [iota + c * LANES for c in range(D_MODEL // LANES)]
        ddst = [(iota + c * LANES) * BB for c in range(D_MODEL // LANES)]

        def out_copies(s, p, wait):
            trans = trans_bufs[p]
            for td in range(8):
                cp = pltpu.make_async_copy(
                    trans.at[pl.ds(td * TILE, TILE)],
                    out_hbm.at[s, td, wid], osems[p])
                if wait:
                    cp.wait()
                else:
                    cp.start()

        for p in range(2):
            pltpu.async_copy(
                tablep_hbm.at[idx_v.at[p]], rows_bufs[p], gsems[p])

        def step(i, carry):
            for p in range(2):
                s = i * 2 + p
                rows = rows_bufs[p]
                trans = trans_bufs[p]
                pltpu.make_async_copy(
                    tablep_hbm.at[idx_v.at[s]], rows, gsems[p]).wait()

                @pl.when(s >= 2)
                def _(s=s, p=p):
                    out_copies(s - 2, p, wait=True)

                @plsc.parallel_loop(0, LANES, unroll=1)
                def _(t, rows=rows, trans=trans):
                    a = (iota + t) & 15
                    for q in range(BB // LANES):
                        bi = a + q * LANES
                        for c in range(D_MODEL // LANES):
                            vals = plsc.load_gather(rows, [bi, dcol[c]])
                            plsc.store_scatter(
                                trans, [ddst[c] + bi], vals * SCALE)

                out_copies(s, p, wait=False)

                @pl.when(s + 2 < S)
                def _(s=s, p=p):
                    pltpu.async_copy(
                        tablep_hbm.at[idx_v.at[s + 2]], rows_bufs[p],
                        gsems[p])

            return carry

        lax.fori_loop(0, S // 2, step, 0)
        for p in range(2):
            out_copies(S - 2 + p, p, wait=True)

    out5 = emb(xt, tablep)
    out5 = out5.reshape(S, 8, NW, 8, BB)
    return out5.transpose(2, 4, 0, 1, 3).reshape(B0, S, D_MODEL)
